# baseline (speedup 1.0000x reference)
import functools

import jax
import jax.numpy as jnp
from jax import lax
from jax.experimental import pallas as pl
from jax.experimental.pallas import tpu as pltpu
from jax.experimental.pallas import tpu_sc as plsc

LS = 0.1
V = 100000
B = 1024
CONF = 1.0 - LS
SMOOTH = LS / (V - 2)
GRAN = 128

SUM_ROWS, SUM_COLS = 3200, 32000
SUM_BLK_ROWS = 64
NBLK = SUM_ROWS // SUM_BLK_ROWS

_SC_WORKERS = 32
_IDX_N = 2 * B
_IDX_PER_W = _IDX_N // _SC_WORKERS


def _sc_gather(flat, idx):
    mesh = plsc.VectorSubcoreMesh(core_axis_name="c", subcore_axis_name="s")

    @functools.partial(
        pl.kernel,
        mesh=mesh,
        out_type=jax.ShapeDtypeStruct((_IDX_N, GRAN), jnp.float32),
        scratch_types=[
            pltpu.VMEM((_IDX_PER_W,), jnp.int32),
            pltpu.VMEM((_IDX_PER_W, GRAN), jnp.float32),
            pltpu.SemaphoreType.DMA,
        ],
    )
    def k(table_hbm, idx_hbm, out_hbm, idx_v, rows_v, sem):
        wid = lax.axis_index("s") * 2 + lax.axis_index("c")
        base = wid * _IDX_PER_W
        pltpu.sync_copy(idx_hbm.at[pl.ds(base, _IDX_PER_W)], idx_v)
        pltpu.async_copy(table_hbm.at[idx_v], rows_v, sem).wait()
        pltpu.sync_copy(rows_v, out_hbm.at[pl.ds(base, _IDX_PER_W)])

    return k(flat, idx)


def _sum_body(x_ref, o_ref):
    o_ref[0, 0, 0] = jnp.sum(x_ref[...])


def _combine_body(p_ref, g_ref, t_ref, o_ref):
    g_t = g_ref[0:B, :]
    g_0 = g_ref[B : 2 * B, :]
    t = t_ref[...]
    brow = jax.lax.broadcasted_iota(jnp.int32, (B, 1), 0)
    lane = jax.lax.broadcasted_iota(jnp.int32, (B, GRAN), 1)
    c_t = jnp.bitwise_and(brow * V + t, GRAN - 1)
    c_0 = jnp.bitwise_and(brow * V, GRAN - 1)
    sel = jnp.sum(jnp.where(lane == c_t, g_t, 0.0), axis=1, keepdims=True)
    sel0 = jnp.sum(jnp.where(lane == c_0, g_0, 0.0), axis=1, keepdims=True)
    t1 = jnp.sum(sel)
    t2 = jnp.sum(jnp.where(t != 0, sel, 0.0))
    c0 = jnp.sum(sel0)
    n0 = jnp.sum(jnp.where(t == 0, 1.0, 0.0))
    g_total = lax.fori_loop(0, NBLK, lambda i, a: a + p_ref[i, 0, 0],
                            jnp.float32(0.0))
    s32 = jnp.float32(SMOOTH)
    conf32 = jnp.float32(CONF)
    const = B * (conf32 * jnp.log(conf32) + (V - 2) * s32 * jnp.log(s32))
    o_ref[0, 0] = (const + n0 * s32 * jnp.log(s32)
                   - s32 * (g_total - c0 - t2) - conf32 * t1)


def kernel(output, target, one_hot):
    del one_hot
    flat = output.reshape(B * V // GRAN, GRAN)
    view = output.reshape(SUM_ROWS, SUM_COLS)
    rowbase = jnp.arange(B, dtype=jnp.int32) * V
    idx = jnp.concatenate([(rowbase + target) // GRAN, rowbase // GRAN])

    gathered = _sc_gather(flat, idx)

    partials = pl.pallas_call(
        _sum_body,
        grid=(NBLK,),
        in_specs=[pl.BlockSpec((SUM_BLK_ROWS, SUM_COLS), lambda j: (j, 0))],
        out_specs=pl.BlockSpec((1, 1, 1), lambda j: (j, 0, 0),
                               memory_space=pltpu.SMEM),
        out_shape=jax.ShapeDtypeStruct((NBLK, 1, 1), jnp.float32),
        compiler_params=pltpu.CompilerParams(dimension_semantics=("parallel",)),
    )(view)

    out = pl.pallas_call(
        _combine_body,
        in_specs=[
            pl.BlockSpec(memory_space=pltpu.SMEM),
            pl.BlockSpec((2 * B, GRAN), lambda: (0, 0)),
            pl.BlockSpec((B, 1), lambda: (0, 0)),
        ],
        out_specs=pl.BlockSpec(memory_space=pltpu.SMEM),
        out_shape=jax.ShapeDtypeStruct((1, 1), jnp.float32),
    )(partials, gathered, target.reshape(B, 1))
    return out[0, 0]


# speedup vs baseline: 2.9870x; 2.9870x over previous
import functools

import jax
import jax.numpy as jnp
from jax import lax
from jax.experimental import pallas as pl
from jax.experimental.pallas import tpu as pltpu
from jax.experimental.pallas import tpu_sc as plsc

LS = 0.1
V = 100000
B = 1024
CONF = 1.0 - LS
SMOOTH = LS / (V - 2)
GRAN = 128
SUB = 8
ALIGNED_LIMIT = (V // GRAN) * GRAN

SUM_BLK_ROWS = 8
NBLK = B // SUM_BLK_ROWS

_ROWS_PER_CORE = B // 2


def _sc_gather(output, starts):
    mesh = plsc.ScalarSubcoreMesh(axis_name="c", num_cores=2)

    @functools.partial(
        pl.kernel,
        mesh=mesh,
        out_type=jax.ShapeDtypeStruct((B, SUB, GRAN), jnp.float32),
        scratch_types=[
            pltpu.SMEM((_ROWS_PER_CORE,), jnp.int32),
            pltpu.SemaphoreType.DMA,
        ],
    )
    def k(out_hbm, st_hbm, g_hbm, st_sm, sem):
        cid = lax.axis_index("c")
        base = cid * _ROWS_PER_CORE
        pltpu.sync_copy(st_hbm.at[pl.ds(base, _ROWS_PER_CORE)], st_sm)

        @pl.loop(0, _ROWS_PER_CORE)
        def _(i):
            b = base + i
            r0 = pl.multiple_of((b // SUB) * SUB, SUB)
            st = pl.multiple_of(st_sm[i], GRAN)
            pltpu.async_copy(
                out_hbm.at[pl.ds(r0, SUB), pl.ds(st, GRAN)],
                g_hbm.at[b], sem,
            )

        @pl.loop(0, _ROWS_PER_CORE)
        def _(i):
            pltpu.make_async_copy(
                out_hbm.at[pl.ds(0, SUB), pl.ds(0, GRAN)],
                g_hbm.at[0], sem,
            ).wait()

    return k(output, starts)


def _sum_body(x_ref, g_ref, c0_ref, tail_ref):
    g_ref[0, 0, 0] = jnp.sum(x_ref[...])
    c0_ref[0, 0, 0] = jnp.sum(x_ref[:, 0:1])
    tail_ref[...] = x_ref[:, V - GRAN:]


def _combine_body(gp_ref, c0p_ref, g_ref, tail_ref, t_ref, st_ref, o_ref):
    t = t_ref[...]
    brow = jax.lax.broadcasted_iota(jnp.int32, (B, 1), 0)
    sub = jnp.bitwise_and(brow, SUB - 1)
    lane = jax.lax.broadcasted_iota(jnp.int32, (B, GRAN), 1)
    sub_iota = jax.lax.broadcasted_iota(jnp.int32, (B, SUB), 1)

    c = t - st_ref[...]
    lane3 = jax.lax.broadcasted_iota(jnp.int32, (B, SUB, GRAN), 2)
    bylane = jnp.sum(jnp.where(lane3 == c[:, :, None], g_ref[...], 0.0), axis=2)
    sel_main = jnp.sum(jnp.where(sub_iota == sub, bylane, 0.0), axis=1,
                       keepdims=True)
    c_tail = t - (V - GRAN)
    sel_tail = jnp.sum(jnp.where(lane == c_tail, tail_ref[...], 0.0), axis=1,
                       keepdims=True)
    sel = jnp.where(t >= ALIGNED_LIMIT, sel_tail, sel_main)

    t1 = jnp.sum(sel)
    t2 = jnp.sum(jnp.where(t != 0, sel, 0.0))
    n0 = jnp.sum(jnp.where(t == 0, 1.0, 0.0))

    def _acc(i, a):
        return a[0] + gp_ref[i, 0, 0], a[1] + c0p_ref[i, 0, 0]

    g_total, c0 = lax.fori_loop(0, NBLK, _acc,
                                (jnp.float32(0.0), jnp.float32(0.0)))
    s32 = jnp.float32(SMOOTH)
    conf32 = jnp.float32(CONF)
    const = B * (conf32 * jnp.log(conf32) + (V - 2) * s32 * jnp.log(s32))
    o_ref[0, 0] = (const + n0 * s32 * jnp.log(s32)
                   - s32 * (g_total - c0 - t2) - conf32 * t1)


def kernel(output, target, one_hot):
    del one_hot
    t0 = (target // GRAN) * GRAN
    starts = jnp.where(t0 >= ALIGNED_LIMIT, 0, t0).astype(jnp.int32)

    gathered = _sc_gather(output, starts)

    gpart, c0part, tail = pl.pallas_call(
        _sum_body,
        grid=(NBLK,),
        in_specs=[pl.BlockSpec((SUM_BLK_ROWS, V), lambda j: (j, 0))],
        out_specs=[
            pl.BlockSpec((1, 1, 1), lambda j: (j, 0, 0),
                         memory_space=pltpu.SMEM),
            pl.BlockSpec((1, 1, 1), lambda j: (j, 0, 0),
                         memory_space=pltpu.SMEM),
            pl.BlockSpec((SUM_BLK_ROWS, GRAN), lambda j: (j, 0)),
        ],
        out_shape=[
            jax.ShapeDtypeStruct((NBLK, 1, 1), jnp.float32),
            jax.ShapeDtypeStruct((NBLK, 1, 1), jnp.float32),
            jax.ShapeDtypeStruct((B, GRAN), jnp.float32),
        ],
        compiler_params=pltpu.CompilerParams(dimension_semantics=("parallel",)),
    )(output)

    out = pl.pallas_call(
        _combine_body,
        in_specs=[
            pl.BlockSpec(memory_space=pltpu.SMEM),
            pl.BlockSpec(memory_space=pltpu.SMEM),
            pl.BlockSpec((B, SUB, GRAN), lambda: (0, 0, 0)),
            pl.BlockSpec((B, GRAN), lambda: (0, 0)),
            pl.BlockSpec((B, 1), lambda: (0, 0)),
            pl.BlockSpec((B, 1), lambda: (0, 0)),
        ],
        out_specs=pl.BlockSpec(memory_space=pltpu.SMEM),
        out_shape=jax.ShapeDtypeStruct((1, 1), jnp.float32),
    )(gpart, c0part, gathered, tail, target.reshape(B, 1),
      starts.reshape(B, 1))
    return out[0, 0]


# speedup vs baseline: 3.3205x; 1.1117x over previous
import functools

import jax
import jax.numpy as jnp
from jax import lax
from jax.experimental import pallas as pl
from jax.experimental.pallas import tpu as pltpu
from jax.experimental.pallas import tpu_sc as plsc

LS = 0.1
V = 100000
B = 1024
CONF = 1.0 - LS
SMOOTH = LS / (V - 2)
GRAN = 128
SUB = 8
ALIGNED_LIMIT = (V // GRAN) * GRAN

SUM_BLK_ROWS = 32
NBLK = B // SUM_BLK_ROWS

_ROWS_PER_CORE = B // 2


def _sc_gather(output, starts):
    mesh = plsc.ScalarSubcoreMesh(axis_name="c", num_cores=2)

    @functools.partial(
        pl.kernel,
        mesh=mesh,
        out_type=jax.ShapeDtypeStruct((B, SUB, GRAN), jnp.float32),
        scratch_types=[
            pltpu.SMEM((_ROWS_PER_CORE,), jnp.int32),
            pltpu.SemaphoreType.DMA,
        ],
    )
    def k(out_hbm, st_hbm, g_hbm, st_sm, sem):
        cid = lax.axis_index("c")
        base = cid * _ROWS_PER_CORE
        pltpu.sync_copy(st_hbm.at[pl.ds(base, _ROWS_PER_CORE)], st_sm)

        @pl.loop(0, _ROWS_PER_CORE)
        def _(i):
            b = base + i
            r0 = pl.multiple_of((b // SUB) * SUB, SUB)
            st = pl.multiple_of(st_sm[i], GRAN)
            pltpu.async_copy(
                out_hbm.at[pl.ds(r0, SUB), pl.ds(st, GRAN)],
                g_hbm.at[b], sem,
            )

        @pl.loop(0, _ROWS_PER_CORE)
        def _(i):
            pltpu.make_async_copy(
                out_hbm.at[pl.ds(0, SUB), pl.ds(0, GRAN)],
                g_hbm.at[0], sem,
            ).wait()

    return k(output, starts)


def _sum_body(x_ref, g_ref, c0_ref, tail_ref):
    g_ref[0, 0, 0] = jnp.sum(x_ref[...])
    c0_ref[0, 0, 0] = jnp.sum(x_ref[:, 0:1])
    tail_ref[...] = x_ref[:, V - GRAN:]


def _combine_body(gp_ref, c0p_ref, g_ref, tail_ref, t_ref, st_ref, o_ref):
    t = t_ref[...]
    brow = jax.lax.broadcasted_iota(jnp.int32, (B, 1), 0)
    sub = jnp.bitwise_and(brow, SUB - 1)
    lane = jax.lax.broadcasted_iota(jnp.int32, (B, GRAN), 1)
    sub_iota = jax.lax.broadcasted_iota(jnp.int32, (B, SUB), 1)

    c = t - st_ref[...]
    lane3 = jax.lax.broadcasted_iota(jnp.int32, (B, SUB, GRAN), 2)
    bylane = jnp.sum(jnp.where(lane3 == c[:, :, None], g_ref[...], 0.0), axis=2)
    sel_main = jnp.sum(jnp.where(sub_iota == sub, bylane, 0.0), axis=1,
                       keepdims=True)
    c_tail = t - (V - GRAN)
    sel_tail = jnp.sum(jnp.where(lane == c_tail, tail_ref[...], 0.0), axis=1,
                       keepdims=True)
    sel = jnp.where(t >= ALIGNED_LIMIT, sel_tail, sel_main)

    t1 = jnp.sum(sel)
    t2 = jnp.sum(jnp.where(t != 0, sel, 0.0))
    n0 = jnp.sum(jnp.where(t == 0, 1.0, 0.0))

    def _acc(i, a):
        return a[0] + gp_ref[i, 0, 0], a[1] + c0p_ref[i, 0, 0]

    g_total, c0 = lax.fori_loop(0, NBLK, _acc,
                                (jnp.float32(0.0), jnp.float32(0.0)))
    s32 = jnp.float32(SMOOTH)
    conf32 = jnp.float32(CONF)
    const = B * (conf32 * jnp.log(conf32) + (V - 2) * s32 * jnp.log(s32))
    o_ref[0, 0] = (const + n0 * s32 * jnp.log(s32)
                   - s32 * (g_total - c0 - t2) - conf32 * t1)


def kernel(output, target, one_hot):
    del one_hot
    t0 = (target // GRAN) * GRAN
    starts = jnp.where(t0 >= ALIGNED_LIMIT, 0, t0).astype(jnp.int32)

    gathered = _sc_gather(output, starts)

    gpart, c0part, tail = pl.pallas_call(
        _sum_body,
        grid=(NBLK,),
        in_specs=[pl.BlockSpec((SUM_BLK_ROWS, V), lambda j: (j, 0))],
        out_specs=[
            pl.BlockSpec((1, 1, 1), lambda j: (j, 0, 0),
                         memory_space=pltpu.SMEM),
            pl.BlockSpec((1, 1, 1), lambda j: (j, 0, 0),
                         memory_space=pltpu.SMEM),
            pl.BlockSpec((SUM_BLK_ROWS, GRAN), lambda j: (j, 0)),
        ],
        out_shape=[
            jax.ShapeDtypeStruct((NBLK, 1, 1), jnp.float32),
            jax.ShapeDtypeStruct((NBLK, 1, 1), jnp.float32),
            jax.ShapeDtypeStruct((B, GRAN), jnp.float32),
        ],
        compiler_params=pltpu.CompilerParams(dimension_semantics=("parallel",)),
    )(output)

    out = pl.pallas_call(
        _combine_body,
        in_specs=[
            pl.BlockSpec(memory_space=pltpu.SMEM),
            pl.BlockSpec(memory_space=pltpu.SMEM),
            pl.BlockSpec((B, SUB, GRAN), lambda: (0, 0, 0)),
            pl.BlockSpec((B, GRAN), lambda: (0, 0)),
            pl.BlockSpec((B, 1), lambda: (0, 0)),
            pl.BlockSpec((B, 1), lambda: (0, 0)),
        ],
        out_specs=pl.BlockSpec(memory_space=pltpu.SMEM),
        out_shape=jax.ShapeDtypeStruct((1, 1), jnp.float32),
    )(gpart, c0part, gathered, tail, target.reshape(B, 1),
      starts.reshape(B, 1))
    return out[0, 0]


# speedup vs baseline: 3.3886x; 1.0205x over previous
import functools

import jax
import jax.numpy as jnp
from jax import lax
from jax.experimental import pallas as pl
from jax.experimental.pallas import tpu as pltpu
from jax.experimental.pallas import tpu_sc as plsc

LS = 0.1
V = 100000
B = 1024
CONF = 1.0 - LS
SMOOTH = LS / (V - 2)
GRAN = 128
SUB = 8

NSTREAM = 4
SUM_BLK_ROWS = 8
NSTEP = B // (SUM_BLK_ROWS * NSTREAM)

_ROWS_PER_CORE = B // 2


def _sc_gather(output, starts):
    mesh = plsc.ScalarSubcoreMesh(axis_name="c", num_cores=2)

    @functools.partial(
        pl.kernel,
        mesh=mesh,
        out_type=jax.ShapeDtypeStruct((B, SUB, GRAN), jnp.float32),
        scratch_types=[
            pltpu.SMEM((_ROWS_PER_CORE,), jnp.int32),
            pltpu.SemaphoreType.DMA,
        ],
    )
    def k(out_hbm, st_hbm, g_hbm, st_sm, sem):
        cid = lax.axis_index("c")
        base = cid * _ROWS_PER_CORE
        pltpu.sync_copy(st_hbm.at[pl.ds(base, _ROWS_PER_CORE)], st_sm)

        @pl.loop(0, _ROWS_PER_CORE)
        def _(i):
            b = base + i
            r0 = pl.multiple_of((b // SUB) * SUB, SUB)
            st = pl.multiple_of(st_sm[i], GRAN)
            pltpu.async_copy(
                out_hbm.at[pl.ds(r0, SUB), pl.ds(st, GRAN)],
                g_hbm.at[b], sem,
            )

        @pl.loop(0, _ROWS_PER_CORE)
        def _(i):
            pltpu.make_async_copy(
                out_hbm.at[pl.ds(0, SUB), pl.ds(0, GRAN)],
                g_hbm.at[0], sem,
            ).wait()

    return k(output, starts)


def _sum_body(x0, x1, x2, x3, g_ref, c0_ref):
    g_ref[0, 0, 0] = (jnp.sum(x0[...]) + jnp.sum(x1[...])
                      + jnp.sum(x2[...]) + jnp.sum(x3[...]))
    c0_ref[0, 0, 0] = (jnp.sum(x0[:, 0:1]) + jnp.sum(x1[:, 0:1])
                       + jnp.sum(x2[:, 0:1]) + jnp.sum(x3[:, 0:1]))


def _combine_body(gp_ref, c0p_ref, g_ref, t_ref, st_ref, o_ref):
    t = t_ref[...]
    brow = jax.lax.broadcasted_iota(jnp.int32, (B, 1), 0)
    sub = jnp.bitwise_and(brow, SUB - 1)
    sub_iota = jax.lax.broadcasted_iota(jnp.int32, (B, SUB), 1)
    lane3 = jax.lax.broadcasted_iota(jnp.int32, (B, SUB, GRAN), 2)

    c = t - st_ref[...]
    bylane = jnp.sum(jnp.where(lane3 == c[:, :, None], g_ref[...], 0.0), axis=2)
    sel = jnp.sum(jnp.where(sub_iota == sub, bylane, 0.0), axis=1,
                  keepdims=True)

    t1 = jnp.sum(sel)
    t2 = jnp.sum(jnp.where(t != 0, sel, 0.0))
    n0 = jnp.sum(jnp.where(t == 0, 1.0, 0.0))

    def _acc(i, a):
        return a[0] + gp_ref[i, 0, 0], a[1] + c0p_ref[i, 0, 0]

    g_total, c0 = lax.fori_loop(0, NSTEP, _acc,
                                (jnp.float32(0.0), jnp.float32(0.0)))
    s32 = jnp.float32(SMOOTH)
    conf32 = jnp.float32(CONF)
    const = B * (conf32 * jnp.log(conf32) + (V - 2) * s32 * jnp.log(s32))
    o_ref[0, 0] = (const + n0 * s32 * jnp.log(s32)
                   - s32 * (g_total - c0 - t2) - conf32 * t1)


def kernel(output, target, one_hot):
    del one_hot
    starts = ((target // GRAN) * GRAN).astype(jnp.int32)

    gathered = _sc_gather(output, starts)

    gpart, c0part = pl.pallas_call(
        _sum_body,
        grid=(NSTEP,),
        in_specs=[
            pl.BlockSpec((SUM_BLK_ROWS, V),
                         functools.partial(lambda k, j: (NSTREAM * j + k, 0), k))
            for k in range(NSTREAM)
        ],
        out_specs=[
            pl.BlockSpec((1, 1, 1), lambda j: (j, 0, 0),
                         memory_space=pltpu.SMEM),
            pl.BlockSpec((1, 1, 1), lambda j: (j, 0, 0),
                         memory_space=pltpu.SMEM),
        ],
        out_shape=[
            jax.ShapeDtypeStruct((NSTEP, 1, 1), jnp.float32),
            jax.ShapeDtypeStruct((NSTEP, 1, 1), jnp.float32),
        ],
        compiler_params=pltpu.CompilerParams(dimension_semantics=("arbitrary",)),
    )(output, output, output, output)

    out = pl.pallas_call(
        _combine_body,
        in_specs=[
            pl.BlockSpec(memory_space=pltpu.SMEM),
            pl.BlockSpec(memory_space=pltpu.SMEM),
            pl.BlockSpec((B, SUB, GRAN), lambda: (0, 0, 0)),
            pl.BlockSpec((B, 1), lambda: (0, 0)),
            pl.BlockSpec((B, 1), lambda: (0, 0)),
        ],
        out_specs=pl.BlockSpec(memory_space=pltpu.SMEM),
        out_shape=jax.ShapeDtypeStruct((1, 1), jnp.float32),
    )(gpart, c0part, gathered, target.reshape(B, 1), starts.reshape(B, 1))
    return out[0, 0]
